# Initial kernel scaffold; baseline (speedup 1.0000x reference)
import jax
import jax.numpy as jnp
from jax.experimental import pallas as pl


def kernel(ipa, type1, time, emb1, emb2, W1, b1, W2, b2, W3, b3, W4, b4):
    raise NotImplementedError("write your pallas kernel here")


# speedup vs baseline: 1.8011x; 1.8011x over previous
import functools

import jax
import jax.numpy as jnp
from jax import lax
from jax.experimental import pallas as pl
from jax.experimental.pallas import tpu as pltpu
from jax.experimental.pallas import tpu_sc as plsc

_IPNUM = 8192
_B = 16384
_D = 16
_NC = 2
_NS = 16
_NW = _NC * _NS
_BPW = _B // _NW
_CH = 128
_NCH = _BPW // _CH

_BS = 2048


def _make_sc_gather():
    mesh = plsc.VectorSubcoreMesh(core_axis_name="c", subcore_axis_name="s")

    @functools.partial(
        pl.kernel,
        mesh=mesh,
        compiler_params=pltpu.CompilerParams(use_tc_tiling_on_sc=False),
        out_type=jax.ShapeDtypeStruct((_B, _D), jnp.float32),
        scratch_types=[
            pltpu.VMEM((_NCH, _CH), jnp.int32),
            pltpu.VMEM((_BPW, _D), jnp.float32),
            pltpu.SemaphoreType.DMA,
        ],
    )
    def sc_gather(table_hbm, idx_hbm, out_hbm, idx_v, rows_v, sem):
        wid = lax.axis_index("s") * _NC + lax.axis_index("c")
        pltpu.sync_copy(idx_hbm.at[wid], idx_v)
        copies = [
            pltpu.async_copy(
                table_hbm.at[idx_v.at[j]],
                rows_v.at[pl.ds(j * _CH, _CH)],
                sem,
            )
            for j in range(_NCH)
        ]
        for c in copies:
            c.wait()
        pltpu.sync_copy(rows_v, out_hbm.at[pl.ds(wid * _BPW, _BPW)])

    return sc_gather


def _mlp_body(rows_ref, t1_ref, time_ref, w1a_ref, e2_ref, w1b_ref,
              w1t_ref, b1_ref, w2_ref, b2_ref, w3_ref, b3_ref,
              w4_ref, b4_ref, out_ref):
    rows = rows_ref[...]
    h = jnp.dot(rows, w1a_ref[...], preferred_element_type=jnp.float32)
    e2w = jnp.dot(e2_ref[...], w1b_ref[...],
                  preferred_element_type=jnp.float32)
    oh = (t1_ref[...] == lax.broadcasted_iota(jnp.int32, (1, 32), 1)
          ).astype(jnp.float32)
    h = h + jnp.dot(oh, e2w, preferred_element_type=jnp.float32)
    h = h + time_ref[...] * w1t_ref[...]
    x = jnp.tanh(h + b1_ref[...])
    x = jnp.tanh(jnp.dot(x, w2_ref[...], preferred_element_type=jnp.float32)
                 + b2_ref[...])
    x = jnp.tanh(jnp.dot(x, w3_ref[...], preferred_element_type=jnp.float32)
                 + b3_ref[...])
    x = jnp.tanh(jnp.dot(x, w4_ref[...], preferred_element_type=jnp.float32)
                 + b4_ref[...])
    out_ref[...] = jax.nn.sigmoid(x[:, 0:1])


def kernel(ipa, type1, time, emb1, emb2, W1, b1, W2, b2, W3, b3, W4, b4):
    f32 = jnp.float32
    table = jnp.zeros((_IPNUM + 1, _D), f32).at[:, :5].set(emb1)
    idx = ipa.reshape(_NW, _NCH, _CH)
    rows = _make_sc_gather()(table, idx)

    w1a = jnp.zeros((_D, 128), f32).at[:5, :20].set(W1[:5])
    w1b = jnp.zeros((_D, 128), f32).at[:5, :20].set(W1[5:10])
    w1t = jnp.zeros((1, 128), f32).at[0, :20].set(W1[10])
    e2p = jnp.zeros((32, _D), f32).at[:21, :5].set(emb2)
    b1p = jnp.zeros((1, 128), f32).at[0, :20].set(b1)
    w2p = jnp.zeros((128, 128), f32).at[:20, :30].set(W2)
    b2p = jnp.zeros((1, 128), f32).at[0, :30].set(b2)
    w3p = jnp.zeros((128, 128), f32).at[:30, :10].set(W3)
    b3p = jnp.zeros((1, 128), f32).at[0, :10].set(b3)
    w4p = jnp.zeros((128, 128), f32).at[:10, :1].set(W4)
    b4p = jnp.zeros((1, 128), f32).at[0, :1].set(b4)

    full = lambda shape: pl.BlockSpec(shape, lambda i: (0, 0))
    out = pl.pallas_call(
        _mlp_body,
        grid=(_B // _BS,),
        in_specs=[
            pl.BlockSpec((_BS, _D), lambda i: (i, 0)),
            pl.BlockSpec((_BS, 1), lambda i: (i, 0)),
            pl.BlockSpec((_BS, 1), lambda i: (i, 0)),
            full((_D, 128)), full((32, _D)), full((_D, 128)),
            full((1, 128)), full((1, 128)),
            full((128, 128)), full((1, 128)),
            full((128, 128)), full((1, 128)),
            full((128, 128)), full((1, 128)),
        ],
        out_specs=pl.BlockSpec((_BS, 1), lambda i: (i, 0)),
        out_shape=jax.ShapeDtypeStruct((_B, 1), f32),
    )(rows, type1, time, w1a, e2p, w1b, w1t, b1p,
      w2p, b2p, w3p, b3p, w4p, b4p)
    return out


# speedup vs baseline: 2.0472x; 1.1367x over previous
import functools

import jax
import jax.numpy as jnp
from jax import lax
from jax.experimental import pallas as pl
from jax.experimental.pallas import tpu as pltpu
from jax.experimental.pallas import tpu_sc as plsc

_IPNUM = 8192
_B = 16384
_D = 8
_NC = 2
_NS = 16
_NW = _NC * _NS
_BPW = _B // _NW
_CH = 128
_NCH = _BPW // _CH

_BS = 2048


def _make_sc_gather():
    mesh = plsc.VectorSubcoreMesh(core_axis_name="c", subcore_axis_name="s")

    @functools.partial(
        pl.kernel,
        mesh=mesh,
        compiler_params=pltpu.CompilerParams(use_tc_tiling_on_sc=False),
        out_type=jax.ShapeDtypeStruct((_B, _D), jnp.float32),
        scratch_types=[
            pltpu.VMEM((_NCH, _CH), jnp.int32),
            pltpu.VMEM((_BPW, _D), jnp.float32),
            pltpu.SemaphoreType.DMA,
        ],
    )
    def sc_gather(table_hbm, idx_hbm, out_hbm, idx_v, rows_v, sem):
        wid = lax.axis_index("s") * _NC + lax.axis_index("c")
        pltpu.sync_copy(idx_hbm.at[wid], idx_v)
        copies = [
            pltpu.async_copy(
                table_hbm.at[idx_v.at[j]],
                rows_v.at[pl.ds(j * _CH, _CH)],
                sem,
            )
            for j in range(_NCH)
        ]
        for c in copies:
            c.wait()
        pltpu.sync_copy(rows_v, out_hbm.at[pl.ds(wid * _BPW, _BPW)])

    return sc_gather


def _mlp_body(rows_ref, t1_ref, time_ref, emb2_ref, w1_ref, b1_ref,
              w2_ref, b2_ref, w3_ref, b3_ref, w4_ref, b4_ref, out_ref):
    f32 = jnp.float32
    rows = rows_ref[...]
    w1 = w1_ref[...]
    h = jnp.dot(rows[:, 0:5], w1[0:5, :], preferred_element_type=f32)
    e2w = jnp.dot(emb2_ref[...], w1[5:10, :],
                  preferred_element_type=f32)
    oh = (t1_ref[...] == lax.broadcasted_iota(jnp.int32, (1, 21), 1)
          ).astype(f32)
    h = h + jnp.dot(oh, e2w, preferred_element_type=f32)
    h = h + time_ref[...] * w1[10:11, :]
    x = jnp.tanh(h + b1_ref[...])
    x = jnp.tanh(jnp.dot(x, w2_ref[...], preferred_element_type=f32)
                 + b2_ref[...])
    x = jnp.tanh(jnp.dot(x, w3_ref[...], preferred_element_type=f32)
                 + b3_ref[...])
    x = jnp.tanh(jnp.dot(x, w4_ref[...], preferred_element_type=f32)
                 + b4_ref[...])
    out_ref[...] = jax.nn.sigmoid(x)


def kernel(ipa, type1, time, emb1, emb2, W1, b1, W2, b2, W3, b3, W4, b4):
    idx = ipa.reshape(_NW, _NCH, _CH)
    table = jnp.zeros((_IPNUM + 1, _D), jnp.float32).at[:, :5].set(emb1)
    rows = _make_sc_gather()(table, idx)

    full2 = lambda a, b: pl.BlockSpec((a, b), lambda i: (0, 0))
    out = pl.pallas_call(
        _mlp_body,
        grid=(_B // _BS,),
        in_specs=[
            pl.BlockSpec((_BS, _D), lambda i: (i, 0)),
            pl.BlockSpec((_BS, 1), lambda i: (i, 0)),
            pl.BlockSpec((_BS, 1), lambda i: (i, 0)),
            full2(21, 5),
            full2(11, 20), full2(1, 20),
            full2(20, 30), full2(1, 30),
            full2(30, 10), full2(1, 10),
            full2(10, 1), full2(1, 1),
        ],
        out_specs=pl.BlockSpec((_BS, 1), lambda i: (i, 0)),
        out_shape=jax.ShapeDtypeStruct((_B, 1), jnp.float32),
    )(rows, type1, time, emb2, W1, b1.reshape(1, 20), W2, b2.reshape(1, 30),
      W3, b3.reshape(1, 10), W4, b4.reshape(1, 1))
    return out


# speedup vs baseline: 3.4391x; 1.6799x over previous
import functools

import jax
import jax.numpy as jnp
from jax import lax
from jax.experimental import pallas as pl
from jax.experimental.pallas import tpu as pltpu
from jax.experimental.pallas import tpu_sc as plsc

_IPNUM = 8192
_B = 16384
_D = 8
_NC = 2
_NS = 16
_NW = _NC * _NS
_BPW = _B // _NW
_CH = 128
_NCH = _BPW // _CH
_L = 16

_BS = 2048


def _make_sc_gather():
    mesh = plsc.VectorSubcoreMesh(core_axis_name="c", subcore_axis_name="s")

    @functools.partial(
        pl.kernel,
        mesh=mesh,
        compiler_params=pltpu.CompilerParams(use_tc_tiling_on_sc=False),
        out_type=jax.ShapeDtypeStruct((5, _B), jnp.float32),
        scratch_types=[
            pltpu.VMEM((_NCH, _CH), jnp.int32),
            pltpu.VMEM((5 * _NCH, _CH), jnp.int32),
            pltpu.VMEM((5, _BPW), jnp.float32),
            pltpu.SemaphoreType.DMA,
        ],
    )
    def sc_gather(tflat_hbm, idx_hbm, out_hbm, idx_v, idxc_v, rt_v, sem):
        wid = lax.axis_index("s") * _NC + lax.axis_index("c")
        base = wid * _BPW
        pltpu.sync_copy(idx_hbm.at[wid], idx_v)
        for j in range(_NCH):
            for k in range(_CH // _L):
                v = idx_v[j, pl.ds(k * _L, _L)]
                v5 = v * 5
                for c in range(5):
                    idxc_v[c * _NCH + j, pl.ds(k * _L, _L)] = v5 + c
        copies = [
            pltpu.async_copy(
                tflat_hbm.at[idxc_v.at[c * _NCH + j]],
                rt_v.at[c, pl.ds(j * _CH, _CH)],
                sem,
            )
            for c in range(5)
            for j in range(_NCH)
        ]
        for cp in copies:
            cp.wait()
        for c in range(5):
            pltpu.sync_copy(rt_v.at[c], out_hbm.at[c, pl.ds(base, _BPW)])

    return sc_gather


def _mlp_body(rows_ref, t1_ref, time_ref, e2t_ref, w1a_ref, w1b_ref,
              w1t_ref, b1_ref, w2t_ref, b2_ref, w3t_ref, b3_ref,
              w4t_ref, b4_ref, out_ref):
    f32 = jnp.float32
    xT = rows_ref[...]
    h = jnp.dot(w1a_ref[...], xT, preferred_element_type=f32)
    e2wT = jnp.dot(w1b_ref[...], e2t_ref[...],
                   preferred_element_type=f32)
    ohT = (lax.broadcasted_iota(jnp.int32, (21, 1), 0) == t1_ref[...]
           ).astype(f32)
    h = h + jnp.dot(e2wT, ohT, preferred_element_type=f32)
    h = h + jnp.dot(w1t_ref[...], time_ref[...], preferred_element_type=f32)
    x = jnp.tanh(h + b1_ref[...])
    x = jnp.tanh(jnp.dot(w2t_ref[...], x, preferred_element_type=f32)
                 + b2_ref[...])
    x = jnp.tanh(jnp.dot(w3t_ref[...], x, preferred_element_type=f32)
                 + b3_ref[...])
    x = jnp.tanh(jnp.dot(w4t_ref[...], x, preferred_element_type=f32)
                 + b4_ref[...])
    out_ref[...] = jax.nn.sigmoid(x)


def kernel(ipa, type1, time, emb1, emb2, W1, b1, W2, b2, W3, b3, W4, b4):
    f32 = jnp.float32
    idx = ipa.reshape(_NW, _NCH, _CH)
    rowsT = _make_sc_gather()(emb1.reshape(-1), idx)

    t1T = type1.reshape(1, _B)
    timeT = time.reshape(1, _B)
    w1a = W1[:5].T
    w1b = W1[5:10].T
    w1t = W1[10:11].T

    full = lambda a, b: pl.BlockSpec((a, b), lambda i: (0, 0))
    out = pl.pallas_call(
        _mlp_body,
        grid=(_B // _BS,),
        in_specs=[
            pl.BlockSpec((5, _BS), lambda i: (0, i)),
            pl.BlockSpec((1, _BS), lambda i: (0, i)),
            pl.BlockSpec((1, _BS), lambda i: (0, i)),
            full(5, 21),
            full(20, 5), full(20, 5), full(20, 1), full(20, 1),
            full(30, 20), full(30, 1),
            full(10, 30), full(10, 1),
            full(1, 10), full(1, 1),
        ],
        out_specs=pl.BlockSpec((1, _BS), lambda i: (0, i)),
        out_shape=jax.ShapeDtypeStruct((1, _B), f32),
    )(rowsT, t1T, timeT, emb2.T, w1a, w1b, w1t, b1.reshape(20, 1),
      W2.T, b2.reshape(30, 1), W3.T, b3.reshape(10, 1),
      W4.T, b4.reshape(1, 1))
    return out.reshape(_B, 1)


# speedup vs baseline: 3.7258x; 1.0834x over previous
import functools

import jax
import jax.numpy as jnp
from jax import lax
from jax.experimental import pallas as pl
from jax.experimental.pallas import tpu as pltpu
from jax.experimental.pallas import tpu_sc as plsc

_IPNUM = 8192
_B = 16384
_D = 8
_NC = 2
_NS = 16
_NW = _NC * _NS
_BPW = _B // _NW
_CH = 128
_NCH = _BPW // _CH
_L = 16

_BS = 4096


def _make_sc_gather():
    mesh = plsc.VectorSubcoreMesh(core_axis_name="c", subcore_axis_name="s")

    @functools.partial(
        pl.kernel,
        mesh=mesh,
        compiler_params=pltpu.CompilerParams(use_tc_tiling_on_sc=False),
        out_type=jax.ShapeDtypeStruct((5, _B), jnp.float32),
        scratch_types=[
            pltpu.VMEM((_NCH, _CH), jnp.int32),
            pltpu.VMEM((5 * _NCH, _CH), jnp.int32),
            pltpu.VMEM((5, _BPW), jnp.float32),
            pltpu.SemaphoreType.DMA,
        ],
    )
    def sc_gather(tflat_hbm, idx_hbm, out_hbm, idx_v, idxc_v, rt_v, sem):
        wid = lax.axis_index("s") * _NC + lax.axis_index("c")
        base = wid * _BPW
        pltpu.sync_copy(idx_hbm.at[wid], idx_v)
        for j in range(_NCH):
            for k in range(_CH // _L):
                v = idx_v[j, pl.ds(k * _L, _L)]
                v5 = v * 5
                for c in range(5):
                    idxc_v[c * _NCH + j, pl.ds(k * _L, _L)] = v5 + c
        copies = [
            pltpu.async_copy(
                tflat_hbm.at[idxc_v.at[c * _NCH + j]],
                rt_v.at[c, pl.ds(j * _CH, _CH)],
                sem,
            )
            for c in range(5)
            for j in range(_NCH)
        ]
        for cp in copies:
            cp.wait()
        for c in range(5):
            pltpu.sync_copy(rt_v.at[c], out_hbm.at[c, pl.ds(base, _BPW)])

    return sc_gather


def _mlp_body(rows_ref, t1_ref, time_ref, e2t_ref, w1a_ref, w1b_ref,
              w1t_ref, b1_ref, w2t_ref, b2_ref, w3t_ref, b3_ref,
              w4t_ref, b4_ref, out_ref):
    f32 = jnp.float32
    xT = rows_ref[...]
    h = jnp.dot(w1a_ref[...], xT, preferred_element_type=f32)
    e2wT = jnp.dot(w1b_ref[...], e2t_ref[...],
                   preferred_element_type=f32)
    ohT = (lax.broadcasted_iota(jnp.int32, (21, 1), 0) == t1_ref[...]
           ).astype(f32)
    h = h + jnp.dot(e2wT, ohT, preferred_element_type=f32)
    h = h + jnp.dot(w1t_ref[...], time_ref[...], preferred_element_type=f32)
    x = jnp.tanh(h + b1_ref[...])
    x = jnp.tanh(jnp.dot(w2t_ref[...], x, preferred_element_type=f32)
                 + b2_ref[...])
    x = jnp.tanh(jnp.dot(w3t_ref[...], x, preferred_element_type=f32)
                 + b3_ref[...])
    x = jnp.tanh(jnp.dot(w4t_ref[...], x, preferred_element_type=f32)
                 + b4_ref[...])
    out_ref[...] = jax.nn.sigmoid(x)


def kernel(ipa, type1, time, emb1, emb2, W1, b1, W2, b2, W3, b3, W4, b4):
    f32 = jnp.float32
    idx = ipa.reshape(_NW, _NCH, _CH)
    rowsT = _make_sc_gather()(emb1.reshape(-1), idx)

    t1T = type1.reshape(1, _B)
    timeT = time.reshape(1, _B)
    w1a = W1[:5].T
    w1b = W1[5:10].T
    w1t = W1[10:11].T

    full = lambda a, b: pl.BlockSpec((a, b), lambda i: (0, 0))
    out = pl.pallas_call(
        _mlp_body,
        grid=(_B // _BS,),
        in_specs=[
            pl.BlockSpec((5, _BS), lambda i: (0, i)),
            pl.BlockSpec((1, _BS), lambda i: (0, i)),
            pl.BlockSpec((1, _BS), lambda i: (0, i)),
            full(5, 21),
            full(20, 5), full(20, 5), full(20, 1), full(20, 1),
            full(30, 20), full(30, 1),
            full(10, 30), full(10, 1),
            full(1, 10), full(1, 1),
        ],
        out_specs=pl.BlockSpec((1, _BS), lambda i: (0, i)),
        out_shape=jax.ShapeDtypeStruct((1, _B), f32),
    )(rowsT, t1T, timeT, emb2.T, w1a, w1b, w1t, b1.reshape(20, 1),
      W2.T, b2.reshape(30, 1), W3.T, b3.reshape(10, 1),
      W4.T, b4.reshape(1, 1))
    return out.reshape(_B, 1)


# speedup vs baseline: 3.8661x; 1.0377x over previous
import functools

import jax
import jax.numpy as jnp
from jax import lax
from jax.experimental import pallas as pl
from jax.experimental.pallas import tpu as pltpu
from jax.experimental.pallas import tpu_sc as plsc

_IPNUM = 8192
_B = 16384
_D = 8
_NC = 2
_NS = 16
_NW = _NC * _NS
_BPW = _B // _NW
_CH = 128
_NCH = _BPW // _CH
_L = 16

_BS = 8192


def _make_sc_gather():
    mesh = plsc.VectorSubcoreMesh(core_axis_name="c", subcore_axis_name="s")

    @functools.partial(
        pl.kernel,
        mesh=mesh,
        compiler_params=pltpu.CompilerParams(use_tc_tiling_on_sc=False),
        out_type=jax.ShapeDtypeStruct((5, _B), jnp.float32),
        scratch_types=[
            pltpu.VMEM((_NCH, _CH), jnp.int32),
            pltpu.VMEM((5 * _NCH, _CH), jnp.int32),
            pltpu.VMEM((5, _BPW), jnp.float32),
            pltpu.SemaphoreType.DMA,
        ],
    )
    def sc_gather(tflat_hbm, idx_hbm, out_hbm, idx_v, idxc_v, rt_v, sem):
        wid = lax.axis_index("s") * _NC + lax.axis_index("c")
        base = wid * _BPW
        pltpu.sync_copy(idx_hbm.at[wid], idx_v)
        for j in range(_NCH):
            for k in range(_CH // _L):
                v = idx_v[j, pl.ds(k * _L, _L)]
                v5 = v * 5
                for c in range(5):
                    idxc_v[c * _NCH + j, pl.ds(k * _L, _L)] = v5 + c
        copies = [
            pltpu.async_copy(
                tflat_hbm.at[idxc_v.at[c * _NCH + j]],
                rt_v.at[c, pl.ds(j * _CH, _CH)],
                sem,
            )
            for c in range(5)
            for j in range(_NCH)
        ]
        for cp in copies:
            cp.wait()
        for c in range(5):
            pltpu.sync_copy(rt_v.at[c], out_hbm.at[c, pl.ds(base, _BPW)])

    return sc_gather


def _mlp_body(rows_ref, t1_ref, time_ref, e2t_ref, w1a_ref, w1b_ref,
              w1t_ref, b1_ref, w2t_ref, b2_ref, w3t_ref, b3_ref,
              w4t_ref, b4_ref, out_ref):
    f32 = jnp.float32
    xT = rows_ref[...]
    h = jnp.dot(w1a_ref[...], xT, preferred_element_type=f32)
    e2wT = jnp.dot(w1b_ref[...], e2t_ref[...],
                   preferred_element_type=f32)
    ohT = (lax.broadcasted_iota(jnp.int32, (21, 1), 0) == t1_ref[...]
           ).astype(f32)
    h = h + jnp.dot(e2wT, ohT, preferred_element_type=f32)
    h = h + jnp.dot(w1t_ref[...], time_ref[...], preferred_element_type=f32)
    x = jnp.tanh(h + b1_ref[...])
    x = jnp.tanh(jnp.dot(w2t_ref[...], x, preferred_element_type=f32)
                 + b2_ref[...])
    x = jnp.tanh(jnp.dot(w3t_ref[...], x, preferred_element_type=f32)
                 + b3_ref[...])
    x = jnp.tanh(jnp.dot(w4t_ref[...], x, preferred_element_type=f32)
                 + b4_ref[...])
    out_ref[...] = jax.nn.sigmoid(x)


def kernel(ipa, type1, time, emb1, emb2, W1, b1, W2, b2, W3, b3, W4, b4):
    f32 = jnp.float32
    idx = ipa.reshape(_NW, _NCH, _CH)
    rowsT = _make_sc_gather()(emb1.reshape(-1), idx)

    t1T = type1.reshape(1, _B)
    timeT = time.reshape(1, _B)
    w1a = W1[:5].T
    w1b = W1[5:10].T
    w1t = W1[10:11].T

    full = lambda a, b: pl.BlockSpec((a, b), lambda i: (0, 0))
    out = pl.pallas_call(
        _mlp_body,
        grid=(_B // _BS,),
        in_specs=[
            pl.BlockSpec((5, _BS), lambda i: (0, i)),
            pl.BlockSpec((1, _BS), lambda i: (0, i)),
            pl.BlockSpec((1, _BS), lambda i: (0, i)),
            full(5, 21),
            full(20, 5), full(20, 5), full(20, 1), full(20, 1),
            full(30, 20), full(30, 1),
            full(10, 30), full(10, 1),
            full(1, 10), full(1, 1),
        ],
        out_specs=pl.BlockSpec((1, _BS), lambda i: (0, i)),
        out_shape=jax.ShapeDtypeStruct((1, _B), f32),
    )(rowsT, t1T, timeT, emb2.T, w1a, w1b, w1t, b1.reshape(20, 1),
      W2.T, b2.reshape(30, 1), W3.T, b3.reshape(10, 1),
      W4.T, b4.reshape(1, 1))
    return out.reshape(_B, 1)


# speedup vs baseline: 3.9413x; 1.0194x over previous
import functools

import jax
import jax.numpy as jnp
from jax import lax
from jax.experimental import pallas as pl
from jax.experimental.pallas import tpu as pltpu
from jax.experimental.pallas import tpu_sc as plsc

_IPNUM = 8192
_B = 16384
_D = 8
_NC = 2
_NS = 16
_NW = _NC * _NS
_BPW = _B // _NW
_CH = 128
_NCH = _BPW // _CH
_L = 16

_BS = 16384


def _make_sc_gather():
    mesh = plsc.VectorSubcoreMesh(core_axis_name="c", subcore_axis_name="s")

    @functools.partial(
        pl.kernel,
        mesh=mesh,
        compiler_params=pltpu.CompilerParams(use_tc_tiling_on_sc=False),
        out_type=jax.ShapeDtypeStruct((5, _B), jnp.float32),
        scratch_types=[
            pltpu.VMEM((_NCH, _CH), jnp.int32),
            pltpu.VMEM((5 * _NCH, _CH), jnp.int32),
            pltpu.VMEM((5, _BPW), jnp.float32),
            pltpu.SemaphoreType.DMA,
        ],
    )
    def sc_gather(tflat_hbm, idx_hbm, out_hbm, idx_v, idxc_v, rt_v, sem):
        wid = lax.axis_index("s") * _NC + lax.axis_index("c")
        base = wid * _BPW
        pltpu.sync_copy(idx_hbm.at[wid], idx_v)
        for j in range(_NCH):
            for k in range(_CH // _L):
                v = idx_v[j, pl.ds(k * _L, _L)]
                v5 = v * 5
                for c in range(5):
                    idxc_v[c * _NCH + j, pl.ds(k * _L, _L)] = v5 + c
        copies = [
            pltpu.async_copy(
                tflat_hbm.at[idxc_v.at[c * _NCH + j]],
                rt_v.at[c, pl.ds(j * _CH, _CH)],
                sem,
            )
            for c in range(5)
            for j in range(_NCH)
        ]
        for cp in copies:
            cp.wait()
        for c in range(5):
            pltpu.sync_copy(rt_v.at[c], out_hbm.at[c, pl.ds(base, _BPW)])

    return sc_gather


def _mlp_body(rows_ref, t1_ref, time_ref, e2t_ref, w1a_ref, w1b_ref,
              w1t_ref, b1_ref, w2t_ref, b2_ref, w3t_ref, b3_ref,
              w4t_ref, b4_ref, out_ref):
    f32 = jnp.float32
    xT = rows_ref[...]
    h = jnp.dot(w1a_ref[...], xT, preferred_element_type=f32)
    e2wT = jnp.dot(w1b_ref[...], e2t_ref[...],
                   preferred_element_type=f32)
    ohT = (lax.broadcasted_iota(jnp.int32, (21, 1), 0) == t1_ref[...]
           ).astype(f32)
    h = h + jnp.dot(e2wT, ohT, preferred_element_type=f32)
    h = h + jnp.dot(w1t_ref[...], time_ref[...], preferred_element_type=f32)
    x = jnp.tanh(h + b1_ref[...])
    x = jnp.tanh(jnp.dot(w2t_ref[...], x, preferred_element_type=f32)
                 + b2_ref[...])
    x = jnp.tanh(jnp.dot(w3t_ref[...], x, preferred_element_type=f32)
                 + b3_ref[...])
    x = jnp.tanh(jnp.dot(w4t_ref[...], x, preferred_element_type=f32)
                 + b4_ref[...])
    out_ref[...] = jax.nn.sigmoid(x)


def kernel(ipa, type1, time, emb1, emb2, W1, b1, W2, b2, W3, b3, W4, b4):
    f32 = jnp.float32
    idx = ipa.reshape(_NW, _NCH, _CH)
    rowsT = _make_sc_gather()(emb1.reshape(-1), idx)

    t1T = type1.reshape(1, _B)
    timeT = time.reshape(1, _B)
    w1a = W1[:5].T
    w1b = W1[5:10].T
    w1t = W1[10:11].T

    full = lambda a, b: pl.BlockSpec((a, b), lambda i: (0, 0))
    out = pl.pallas_call(
        _mlp_body,
        grid=(_B // _BS,),
        in_specs=[
            pl.BlockSpec((5, _BS), lambda i: (0, i)),
            pl.BlockSpec((1, _BS), lambda i: (0, i)),
            pl.BlockSpec((1, _BS), lambda i: (0, i)),
            full(5, 21),
            full(20, 5), full(20, 5), full(20, 1), full(20, 1),
            full(30, 20), full(30, 1),
            full(10, 30), full(10, 1),
            full(1, 10), full(1, 1),
        ],
        out_specs=pl.BlockSpec((1, _BS), lambda i: (0, i)),
        out_shape=jax.ShapeDtypeStruct((1, _B), f32),
    )(rowsT, t1T, timeT, emb2.T, w1a, w1b, w1t, b1.reshape(20, 1),
      W2.T, b2.reshape(30, 1), W3.T, b3.reshape(10, 1),
      W4.T, b4.reshape(1, 1))
    return out.reshape(_B, 1)
